# Initial kernel scaffold; baseline (speedup 1.0000x reference)
import jax
import jax.numpy as jnp
from jax.experimental import pallas as pl


def kernel(stimulus_set, percept_gate, kernel_gate, table0, table1, w0, w1):
    raise NotImplementedError("write your pallas kernel here")


# speedup vs baseline: 8.8273x; 8.8273x over previous
import jax
import jax.numpy as jnp
from jax import lax
from jax.experimental import pallas as pl
from jax.experimental.pallas import tpu as pltpu
from jax.experimental.pallas import tpu_sc as plsc

B = 16384
N_REF = 4
LANES = 16

_NC = 2
_NS = 16
NW = _NC * _NS
ROWS = B // NW
GROUPS = ROWS // LANES


def _sqrt16(x):
    i = plsc.bitcast(x, jnp.int32)
    i = jnp.int32(0x5F3759DF) - (i >> 1)
    y = plsc.bitcast(i, jnp.float32)
    for _ in range(3):
        y = y * (1.5 - 0.5 * x * y * y)
    return x * y


def _body(stim_hbm, pg_hbm, kg_hbm, t0_hbm, t1_hbm, w_hbm, out_hbm,
          stim_v, pg_v, kg_v, t0_v, t1_v, w_v, out_v):
    wid = lax.axis_index("s") * _NC + lax.axis_index("c")
    base = wid * ROWS
    pltpu.sync_copy(stim_hbm.at[pl.ds(base * 5, ROWS * 5)], stim_v)
    pltpu.sync_copy(pg_hbm.at[pl.ds(base * 2, ROWS * 2)], pg_v)
    pltpu.sync_copy(kg_hbm.at[pl.ds(base * 2, ROWS * 2)], kg_v)
    pltpu.sync_copy(t0_hbm, t0_v)
    pltpu.sync_copy(t1_hbm, t1_v)
    pltpu.sync_copy(w_hbm, w_v)

    w00 = w_v[pl.ds(0 * LANES, LANES)]
    w01 = w_v[pl.ds(1 * LANES, LANES)]
    w10 = w_v[pl.ds(2 * LANES, LANES)]
    w11 = w_v[pl.ds(3 * LANES, LANES)]

    iota = lax.iota(jnp.int32, LANES)
    iota5 = iota * 5
    iota2 = iota * 2
    iota4 = iota * 4

    def group(g, carry):
        r5 = iota5 + g * (LANES * 5)
        r2 = iota2 + g * (LANES * 2)
        r4 = iota4 + g * (LANES * 4)
        pg0 = plsc.load_gather(pg_v, [r2])
        pg1 = plsc.load_gather(pg_v, [r2 + 1])
        kg0 = plsc.load_gather(kg_v, [r2])
        kg1 = plsc.load_gather(kg_v, [r2 + 1])
        z = []
        for s in range(5):
            idx2 = plsc.load_gather(stim_v, [r5 + s]) * 2
            z0d0 = plsc.load_gather(t0_v, [idx2])
            z0d1 = plsc.load_gather(t0_v, [idx2 + 1])
            z1d0 = plsc.load_gather(t1_v, [idx2])
            z1d1 = plsc.load_gather(t1_v, [idx2 + 1])
            z.append((pg0 * z0d0 + pg1 * z1d0, pg0 * z0d1 + pg1 * z1d1))
        sv = []
        for r in range(1, 5):
            dd0 = z[0][0] - z[r][0]
            dd1 = z[0][1] - z[r][1]
            q0 = dd0 * dd0
            q1 = dd1 * dd1
            d0 = _sqrt16(w00 * q0 + w01 * q1 + 1e-12)
            d1 = _sqrt16(w10 * q0 + w11 * q1 + 1e-12)
            s0 = jnp.exp(-10.0 * d0)
            s1 = jnp.exp(-10.0 * d1)
            sv.append(kg0 * s0 + kg1 * s1)
        inv = 1.0 / (sv[0] + sv[1] + sv[2] + sv[3])
        for r in range(N_REF):
            plsc.store_scatter(out_v, [r4 + r], sv[r] * inv)
        return carry

    lax.fori_loop(0, GROUPS, group, 0)
    pltpu.sync_copy(out_v, out_hbm.at[pl.ds(base * N_REF, ROWS * N_REF)])


_sc_call = pl.kernel(
    _body,
    out_type=jax.ShapeDtypeStruct((B * N_REF,), jnp.float32),
    mesh=plsc.VectorSubcoreMesh(core_axis_name="c", subcore_axis_name="s"),
    compiler_params=pltpu.CompilerParams(
        needs_layout_passes=False, use_tc_tiling_on_sc=False),
    scratch_types=[
        pltpu.VMEM((ROWS * 5,), jnp.int32),
        pltpu.VMEM((ROWS * 2,), jnp.float32),
        pltpu.VMEM((ROWS * 2,), jnp.float32),
        pltpu.VMEM((64,), jnp.float32),
        pltpu.VMEM((64,), jnp.float32),
        pltpu.VMEM((4 * LANES,), jnp.float32),
        pltpu.VMEM((ROWS * N_REF,), jnp.float32),
    ],
)


def kernel(stimulus_set, percept_gate, kernel_gate, table0, table1, w0, w1):
    stim = stimulus_set.astype(jnp.int32).reshape(-1)
    pg = percept_gate.reshape(-1)
    kg = kernel_gate.reshape(-1)
    t0 = jnp.zeros((64,), jnp.float32).at[:62].set(table0.reshape(-1))
    t1 = jnp.zeros((64,), jnp.float32).at[:62].set(table1.reshape(-1))
    wmat = jnp.concatenate(
        [jnp.broadcast_to(w0[:, None], (2, LANES)),
         jnp.broadcast_to(w1[:, None], (2, LANES))], axis=0).reshape(-1)
    out = _sc_call(stim, pg, kg, t0, t1, wmat)
    return out.reshape(B, N_REF)
